# baseline (speedup 1.0000x reference)
import functools

import jax
import jax.numpy as jnp
from jax import lax
from jax.experimental import pallas as pl
from jax.experimental.pallas import tpu as pltpu
from jax.experimental.pallas import tpu_sc as plsc

BATCH = 4096
SEQ_LEN = 200
EMBED_DIM = 64

_info = plsc.get_sparse_core_info()
NC, NS = _info.num_cores, _info.num_subcores
NW = NC * NS
B_PER_W = BATCH // NW
NBUF = 2


def _gather_kernel(table_hbm, idx_hbm, out_hbm,
                   idx_v, rows_v, blk_v,
                   gs0, gs1, os0, os1):
    gsem = (gs0, gs1)
    osem = (os0, os1)
    wid = lax.axis_index("s") * NC + lax.axis_index("c")
    b0 = wid * B_PER_W

    pltpu.sync_copy(idx_hbm.at[:, pl.ds(b0, B_PER_W)], idx_v)

    def gather_desc(l, k):
        return pltpu.make_async_copy(
            table_hbm.at[idx_v.at[l]], rows_v.at[k], gsem[k]
        )

    def oc_desc(l, kb):
        return pltpu.make_async_copy(
            blk_v.at[kb], out_hbm.at[l, :, pl.ds(wid, 1)], osem[kb]
        )

    def transpose_fast(kin, kout):
        rv = rows_v.at[kin]
        for jg in range(8):
            j0 = jg * 16
            jvec = lax.iota(jnp.int32, 16) + j0
            for R in range(8):
                for r in range(8):
                    d = 8 * R + r
                    dvec = jnp.full((16,), 0, jnp.int32) + d
                    vals = plsc.load_gather(rv, [jvec, dvec])
                    blk_v[kout, R, 0, r, pl.ds(j0, 16)] = vals

    def transpose_slow(kin, kout):
        rv = rows_v.at[kin]

        def jbody(jg, _):
            j0 = jg * 16
            jvec = lax.iota(jnp.int32, 16) + j0

            def rbody(R, _):
                for r in range(8):
                    dvec = jnp.full((16,), 0, jnp.int32) + (8 * R + r)
                    vals = plsc.load_gather(rv, [jvec, dvec])
                    blk_v[kout, R, 0, r, pl.ds(j0, 16)] = vals
                return ()

            lax.fori_loop(0, 8, rbody, ())
            return ()

        lax.fori_loop(0, 8, jbody, ())

    def step(l, k, fast=True, first=False, pf=True):
        gather_desc(l, k).wait()
        if pf:
            gather_desc(l + 1, 1 - k).start()
        if fast:
            transpose_fast(k, k)
        else:
            transpose_slow(k, k)
        if not first:
            oc_desc(l - 1, 1 - k).wait()
        oc_desc(l, k).start()

    gather_desc(0, 0).start()
    step(0, 0, fast=False, first=True)

    def body(t, _):
        l = 2 * t + 1
        step(l, 1)
        step(l + 1, 0)
        return ()

    lax.fori_loop(0, (SEQ_LEN - 2) // 2, body, ())

    step(SEQ_LEN - 1, 1, fast=False, pf=False)
    oc_desc(SEQ_LEN - 1, 1).wait()


@jax.jit
def _run(table, idx_t):
    mesh = plsc.VectorSubcoreMesh(core_axis_name="c", subcore_axis_name="s")
    fn = functools.partial(
        pl.kernel,
        mesh=mesh,
        out_type=jax.ShapeDtypeStruct((SEQ_LEN, 8, NW, 8, 128), jnp.float32),
        scratch_types=[
            pltpu.VMEM((SEQ_LEN, B_PER_W), jnp.int32),
            pltpu.VMEM((NBUF, B_PER_W, EMBED_DIM), jnp.float32),
            pltpu.VMEM((2, 8, 1, 8, 128), jnp.float32),
            pltpu.SemaphoreType.DMA,
            pltpu.SemaphoreType.DMA,
            pltpu.SemaphoreType.DMA,
            pltpu.SemaphoreType.DMA,
        ],
        compiler_params=pltpu.CompilerParams(
            use_tc_tiling_on_sc=False,
            needs_layout_passes=False,
            disable_bounds_checks=True,
        ),
    )(_gather_kernel)
    return fn(table, idx_t)


def kernel(indices, table):
    idx_t = jnp.swapaxes(indices, 0, 1).astype(jnp.int32)
    out = _run(table, idx_t)
    return jnp.transpose(out, (2, 4, 0, 1, 3)).reshape(BATCH, SEQ_LEN, EMBED_DIM)


# speedup vs baseline: 1.6242x; 1.6242x over previous
import functools

import jax
import jax.numpy as jnp
from jax import lax
from jax.experimental import pallas as pl
from jax.experimental.pallas import tpu as pltpu
from jax.experimental.pallas import tpu_sc as plsc

BATCH = 4096
SEQ_LEN = 200
EMBED_DIM = 64
NUM_ROWS = BATCH * SEQ_LEN

_info = plsc.get_sparse_core_info()
NC, NS = _info.num_cores, _info.num_subcores
NW = NC * NS
ROWS_PER_W = NUM_ROWS // NW
CHUNK = 512
CHUNKS_PER_W = ROWS_PER_W // CHUNK


def _gather_kernel(table_hbm, idx_hbm, out_hbm, idx_v, rows_v, gs0, gs1, os0, os1):
    gsem = (gs0, gs1)
    osem = (os0, os1)
    wid = lax.axis_index("s") * NC + lax.axis_index("c")
    base = wid * ROWS_PER_W
    pltpu.sync_copy(idx_hbm.at[pl.ds(base, ROWS_PER_W)], idx_v)

    def gather_desc(j, b):
        return pltpu.make_async_copy(
            table_hbm.at[idx_v.at[pl.ds(j * CHUNK, CHUNK)]], rows_v.at[b], gsem[b]
        )

    def oc_desc(j, b):
        return pltpu.make_async_copy(
            rows_v.at[b], out_hbm.at[pl.ds(base + j * CHUNK, CHUNK)], osem[b]
        )

    gather_desc(0, 0).start()
    gather_desc(0, 0).wait()
    oc_desc(0, 0).start()
    gather_desc(1, 1).start()

    def body(t, _):
        j = 2 * t + 1
        gather_desc(j, 1).wait()
        oc_desc(j, 1).start()
        oc_desc(j - 1, 0).wait()
        gather_desc(j + 1, 0).start()

        j2 = j + 1
        gather_desc(j2, 0).wait()
        oc_desc(j2, 0).start()
        oc_desc(j2 - 1, 1).wait()
        gather_desc(j2 + 1, 1).start()
        return ()

    lax.fori_loop(0, (CHUNKS_PER_W - 2) // 2, body, ())

    jl = CHUNKS_PER_W - 1
    gather_desc(jl, 1).wait()
    oc_desc(jl, 1).start()
    oc_desc(jl - 1, 0).wait()
    oc_desc(jl, 1).wait()


@jax.jit
def _run(table, idx_flat):
    mesh = plsc.VectorSubcoreMesh(core_axis_name="c", subcore_axis_name="s")
    fn = functools.partial(
        pl.kernel,
        mesh=mesh,
        out_type=jax.ShapeDtypeStruct((NUM_ROWS, EMBED_DIM), jnp.float32),
        scratch_types=[
            pltpu.VMEM((ROWS_PER_W,), jnp.int32),
            pltpu.VMEM((2, CHUNK, EMBED_DIM), jnp.float32),
            pltpu.SemaphoreType.DMA,
            pltpu.SemaphoreType.DMA,
            pltpu.SemaphoreType.DMA,
            pltpu.SemaphoreType.DMA,
        ],
        compiler_params=pltpu.CompilerParams(use_tc_tiling_on_sc=False),
    )(_gather_kernel)
    return fn(table, idx_flat)


def kernel(indices, table):
    idx_flat = indices.reshape(-1).astype(jnp.int32)
    out = _run(table, idx_flat)
    return out.reshape(BATCH, SEQ_LEN, EMBED_DIM)
